# Initial kernel scaffold; baseline (speedup 1.0000x reference)
import jax
import jax.numpy as jnp
from jax.experimental import pallas as pl


def kernel(x, e, g, W_e, b_e, W_n, b_n, W_g, b_g, edges, node_idx, edge_idx):
    raise NotImplementedError("write your pallas kernel here")


# speedup vs baseline: 4.6518x; 4.6518x over previous
import functools
import jax
import jax.numpy as jnp
from jax import lax
from jax.experimental import pallas as pl
from jax.experimental.pallas import tpu as pltpu
from jax.experimental.pallas import tpu_sc as plsc

_N_NODES = 10000
_N_EDGES = 320000
_D = 128
_NG = 16
_CH = 80
_NCHUNKS = _N_EDGES // _CH
_NWORKERS = 32
_CH_PER_W = _NCHUNKS // _NWORKERS
_STRIPE = 624
_STRIPE_TAIL = _N_NODES - 16 * _STRIPE


def _proj_nodes_body(x_ref, wes_ref, wed_ref, ps_ref, pd_ref):
    xb = x_ref[...]
    ps_ref[...] = jnp.dot(xb, wes_ref[...], preferred_element_type=jnp.float32)
    pd_ref[...] = jnp.dot(xb, wed_ref[...], preferred_element_type=jnp.float32)


def _proj_nodes(x, W_es, W_ed):
    blk = 1000
    grid = _N_NODES // blk
    return pl.pallas_call(
        _proj_nodes_body,
        grid=(grid,),
        in_specs=[
            pl.BlockSpec((blk, _D), lambda i: (i, 0)),
            pl.BlockSpec((_D, _D), lambda i: (0, 0)),
            pl.BlockSpec((_D, _D), lambda i: (0, 0)),
        ],
        out_specs=[
            pl.BlockSpec((blk, _D), lambda i: (i, 0)),
            pl.BlockSpec((blk, _D), lambda i: (i, 0)),
        ],
        out_shape=[
            jax.ShapeDtypeStruct((_N_NODES, _D), jnp.float32),
            jax.ShapeDtypeStruct((_N_NODES, _D), jnp.float32),
        ],
    )(x, W_es, W_ed)


def _proj_edges_body(e_ref, wee_ref, g_ref, weg_ref, be_ref, eidx_ref,
                     pe_ref, pg_scr):
    @pl.when(pl.program_id(0) == 0)
    def _():
        pg_scr[...] = (jnp.dot(g_ref[...], weg_ref[...],
                               preferred_element_type=jnp.float32)
                       + be_ref[...])

    eidx = eidx_ref[0, 0, :]
    oh = (eidx[:, None] == lax.broadcasted_iota(jnp.int32, (eidx.shape[0], _NG), 1)
          ).astype(jnp.float32)
    pe_ref[...] = (jnp.dot(e_ref[...], wee_ref[...],
                           preferred_element_type=jnp.float32)
                   + jnp.dot(oh, pg_scr[...], preferred_element_type=jnp.float32))


def _proj_edges(e, W_ee, g, W_eg, b_e2d, edge_idx3d):
    blk = 2000
    grid = _N_EDGES // blk
    return pl.pallas_call(
        _proj_edges_body,
        grid=(grid,),
        in_specs=[
            pl.BlockSpec((blk, 16), lambda i: (i, 0)),
            pl.BlockSpec((16, _D), lambda i: (0, 0)),
            pl.BlockSpec((_NG, 64), lambda i: (0, 0)),
            pl.BlockSpec((64, _D), lambda i: (0, 0)),
            pl.BlockSpec((1, _D), lambda i: (0, 0)),
            pl.BlockSpec((1, 1, blk), lambda i: (i, 0, 0)),
        ],
        out_specs=pl.BlockSpec((blk, _D), lambda i: (i, 0)),
        out_shape=jax.ShapeDtypeStruct((_N_EDGES, _D), jnp.float32),
        scratch_shapes=[pltpu.VMEM((_NG, _D), jnp.float32)],
    )(e, W_ee, g, W_eg, b_e2d, edge_idx3d)


def _sc_edge_body(psrc, pdst, pe, srcidx, dstidx, gidx,
                  eattr, agg0, agg1, eagg0, eagg1,
                  src_v, dst_v, gid_v, s_v, d_v, out_v,
                  agg_sh, eagg_sh, sem_a, sem_b):
    cid = lax.axis_index("c")
    sid = lax.axis_index("s")
    wid = cid * 16 + sid

    def _zero_row(i, _):
        for c in range(_D // 16):
            out_v[i, pl.ds(c * 16, 16)] = jnp.zeros((16,), jnp.float32)
        return 0
    lax.fori_loop(0, _CH, _zero_row, 0)
    for j, cnt in enumerate((80, 80, 80, 80, 80, 80, 80, 64)):
        pltpu.sync_copy(out_v.at[pl.ds(0, cnt)],
                        agg_sh.at[pl.ds(sid * _STRIPE + j * 80, cnt)])

    @pl.when(sid == 0)
    def _():
        pltpu.sync_copy(out_v.at[pl.ds(0, _STRIPE_TAIL)],
                        agg_sh.at[pl.ds(16 * _STRIPE, _STRIPE_TAIL)])
        pltpu.sync_copy(out_v.at[pl.ds(0, _NG)], eagg_sh)

    plsc.subcore_barrier()

    start = wid * _CH_PER_W

    def _chunk(t, _):
        off = (start + t) * _CH
        c1 = pltpu.async_copy(srcidx.at[pl.ds(off, _CH)], src_v, sem_a)
        c2 = pltpu.async_copy(dstidx.at[pl.ds(off, _CH)], dst_v, sem_a)
        c3 = pltpu.async_copy(gidx.at[pl.ds(off, _CH)], gid_v, sem_a)
        c4 = pltpu.async_copy(pe.at[pl.ds(off, _CH)], out_v, sem_a)
        c1.wait(); c2.wait(); c3.wait(); c4.wait()
        g1 = pltpu.async_copy(psrc.at[src_v], s_v, sem_b)
        g2 = pltpu.async_copy(pdst.at[dst_v], d_v, sem_b)
        g1.wait(); g2.wait()

        def _row(i, _):
            for c in range(_D // 16):
                sl = pl.ds(c * 16, 16)
                v = out_v[i, sl] + s_v[i, sl] + d_v[i, sl]
                out_v[i, sl] = jnp.maximum(v, 0.0)
            return 0
        lax.fori_loop(0, _CH, _row, 0)

        pltpu.sync_copy(out_v, eattr.at[pl.ds(off, _CH)])
        pltpu.sync_copy(out_v, agg_sh.at[dst_v], add=True)
        pltpu.sync_copy(out_v, eagg_sh.at[gid_v], add=True)
        return 0

    lax.fori_loop(0, _CH_PER_W, _chunk, 0)
    plsc.subcore_barrier()

    rows = pl.ds(sid * _STRIPE, _STRIPE)
    tail = pl.ds(16 * _STRIPE, _STRIPE_TAIL)

    @pl.when(cid == 0)
    def _():
        pltpu.sync_copy(agg_sh.at[rows], agg0.at[rows])

    @pl.when(cid == 1)
    def _():
        pltpu.sync_copy(agg_sh.at[rows], agg1.at[rows])

    @pl.when((cid == 0) & (sid == 0))
    def _():
        pltpu.sync_copy(agg_sh.at[tail], agg0.at[tail])
        pltpu.sync_copy(eagg_sh, eagg0)

    @pl.when((cid == 1) & (sid == 0))
    def _():
        pltpu.sync_copy(agg_sh.at[tail], agg1.at[tail])
        pltpu.sync_copy(eagg_sh, eagg1)


def _sc_edge(P_src, P_dst, P_e, src, dst, edge_idx):
    mesh = plsc.VectorSubcoreMesh(core_axis_name="c", subcore_axis_name="s")
    f = pl.kernel(
        _sc_edge_body,
        mesh=mesh,
        out_type=[
            jax.ShapeDtypeStruct((_N_EDGES, _D), jnp.float32),
            jax.ShapeDtypeStruct((_N_NODES, _D), jnp.float32),
            jax.ShapeDtypeStruct((_N_NODES, _D), jnp.float32),
            jax.ShapeDtypeStruct((_NG, _D), jnp.float32),
            jax.ShapeDtypeStruct((_NG, _D), jnp.float32),
        ],
        scratch_types=[
            pltpu.VMEM((_CH,), jnp.int32),
            pltpu.VMEM((_CH,), jnp.int32),
            pltpu.VMEM((_CH,), jnp.int32),
            pltpu.VMEM((_CH, _D), jnp.float32),
            pltpu.VMEM((_CH, _D), jnp.float32),
            pltpu.VMEM((_CH, _D), jnp.float32),
            pltpu.VMEM_SHARED((_N_NODES, _D), jnp.float32),
            pltpu.VMEM_SHARED((_NG, _D), jnp.float32),
            pltpu.SemaphoreType.DMA,
            pltpu.SemaphoreType.DMA,
        ],
    )
    return f(P_src, P_dst, P_e, src, dst, edge_idx)


def _node_body(x_ref, a0_ref, a1_ref, nidx_ref, wn1_ref, wn2_ref,
               g_ref, wn3_ref, bn_ref, e0_ref, e1_ref,
               wg1_ref, wg2_ref, wg3_ref, bg_ref,
               na_ref, ga_ref, gn_scr, gacc_scr):
    i = pl.program_id(0)
    nsteps = pl.num_programs(0)

    @pl.when(i == 0)
    def _():
        gn_scr[...] = (jnp.dot(g_ref[...], wn3_ref[...],
                               preferred_element_type=jnp.float32)
                       + bn_ref[...])
        gacc_scr[...] = jnp.zeros_like(gacc_scr)

    nidx = nidx_ref[0, 0, :]
    nb = nidx.shape[0]
    oh = (nidx[:, None] == lax.broadcasted_iota(jnp.int32, (nb, _NG), 1)
          ).astype(jnp.float32)
    oht = (nidx[None, :] == lax.broadcasted_iota(jnp.int32, (_NG, nb), 0)
           ).astype(jnp.float32)
    agg = a0_ref[...] + a1_ref[...]
    na = (jnp.dot(x_ref[...], wn1_ref[...], preferred_element_type=jnp.float32)
          + jnp.dot(agg, wn2_ref[...], preferred_element_type=jnp.float32)
          + jnp.dot(oh, gn_scr[...], preferred_element_type=jnp.float32))
    na = jnp.maximum(na, 0.0)
    na_ref[...] = na
    gacc_scr[...] += jnp.dot(oht, na, preferred_element_type=jnp.float32)

    @pl.when(i == nsteps - 1)
    def _():
        eagg = e0_ref[...] + e1_ref[...]
        ga_ref[...] = (
            jnp.dot(g_ref[...], wg1_ref[...], preferred_element_type=jnp.float32)
            + jnp.dot(gacc_scr[...], wg2_ref[...], preferred_element_type=jnp.float32)
            + jnp.dot(eagg, wg3_ref[...], preferred_element_type=jnp.float32)
            + bg_ref[...])


def _node_global(x, agg0, agg1, nidx3d, W_n1, W_n2, g, W_n3, b_n2d,
                 eagg0, eagg1, W_g1, W_g2, W_g3, b_g2d):
    blk = 1000
    grid = _N_NODES // blk
    return pl.pallas_call(
        _node_body,
        grid=(grid,),
        in_specs=[
            pl.BlockSpec((blk, _D), lambda i: (i, 0)),
            pl.BlockSpec((blk, _D), lambda i: (i, 0)),
            pl.BlockSpec((blk, _D), lambda i: (i, 0)),
            pl.BlockSpec((1, 1, blk), lambda i: (i, 0, 0)),
            pl.BlockSpec((_D, _D), lambda i: (0, 0)),
            pl.BlockSpec((_D, _D), lambda i: (0, 0)),
            pl.BlockSpec((_NG, 64), lambda i: (0, 0)),
            pl.BlockSpec((64, _D), lambda i: (0, 0)),
            pl.BlockSpec((1, _D), lambda i: (0, 0)),
            pl.BlockSpec((_NG, _D), lambda i: (0, 0)),
            pl.BlockSpec((_NG, _D), lambda i: (0, 0)),
            pl.BlockSpec((64, 64), lambda i: (0, 0)),
            pl.BlockSpec((_D, 64), lambda i: (0, 0)),
            pl.BlockSpec((_D, 64), lambda i: (0, 0)),
            pl.BlockSpec((1, 64), lambda i: (0, 0)),
        ],
        out_specs=[
            pl.BlockSpec((blk, _D), lambda i: (i, 0)),
            pl.BlockSpec((_NG, 64), lambda i: (0, 0)),
        ],
        out_shape=[
            jax.ShapeDtypeStruct((_N_NODES, _D), jnp.float32),
            jax.ShapeDtypeStruct((_NG, 64), jnp.float32),
        ],
        scratch_shapes=[
            pltpu.VMEM((_NG, _D), jnp.float32),
            pltpu.VMEM((_NG, _D), jnp.float32),
        ],
    )(x, agg0, agg1, nidx3d, W_n1, W_n2, g, W_n3, b_n2d,
      eagg0, eagg1, W_g1, W_g2, W_g3, b_g2d)


@jax.jit
def kernel(x, e, g, W_e, b_e, W_n, b_n, W_g, b_g, edges, node_idx, edge_idx):
    src = edges[0].astype(jnp.int32)
    dst = edges[1].astype(jnp.int32)
    edge_idx = edge_idx.astype(jnp.int32)
    node_idx = node_idx.astype(jnp.int32)

    W_ee, W_es, W_ed, W_eg = W_e[:16], W_e[16:144], W_e[144:272], W_e[272:336]
    W_n1, W_n2, W_n3 = W_n[:128], W_n[128:256], W_n[256:320]
    W_g1, W_g2, W_g3 = W_g[:64], W_g[64:192], W_g[192:320]

    P_src, P_dst = _proj_nodes(x, W_es, W_ed)
    P_e = _proj_edges(e, W_ee, g, W_eg, b_e.reshape(1, _D),
                      edge_idx.reshape(_N_EDGES // 2000, 1, 2000))
    edge_attr, agg0, agg1, eagg0, eagg1 = _sc_edge(P_src, P_dst, P_e,
                                                   src, dst, edge_idx)
    node_attr, global_attr = _node_global(
        x, agg0, agg1, node_idx.reshape(_N_NODES // 1000, 1, 1000),
        W_n1, W_n2, g, W_n3, b_n.reshape(1, _D),
        eagg0, eagg1, W_g1, W_g2, W_g3, b_g.reshape(1, 64))
    return (edge_attr, node_attr, global_attr)


# speedup vs baseline: 5.1231x; 1.1013x over previous
import functools
import jax
import jax.numpy as jnp
from jax import lax
from jax.experimental import pallas as pl
from jax.experimental.pallas import tpu as pltpu
from jax.experimental.pallas import tpu_sc as plsc

_N_NODES = 10000
_N_EDGES = 320000
_D = 128
_NG = 16
_CH = 40
_NCHUNKS = _N_EDGES // _CH
_NWORKERS = 32
_CH_PER_W = _NCHUNKS // _NWORKERS
_STRIPE = 624
_STRIPE_TAIL = _N_NODES - 16 * _STRIPE


def _proj_nodes_body(x_ref, wes_ref, wed_ref, ps_ref, pd_ref):
    xb = x_ref[...]
    ps_ref[...] = jnp.dot(xb, wes_ref[...], preferred_element_type=jnp.float32)
    pd_ref[...] = jnp.dot(xb, wed_ref[...], preferred_element_type=jnp.float32)


def _proj_nodes(x, W_es, W_ed):
    blk = 1000
    grid = _N_NODES // blk
    return pl.pallas_call(
        _proj_nodes_body,
        grid=(grid,),
        in_specs=[
            pl.BlockSpec((blk, _D), lambda i: (i, 0)),
            pl.BlockSpec((_D, _D), lambda i: (0, 0)),
            pl.BlockSpec((_D, _D), lambda i: (0, 0)),
        ],
        out_specs=[
            pl.BlockSpec((blk, _D), lambda i: (i, 0)),
            pl.BlockSpec((blk, _D), lambda i: (i, 0)),
        ],
        out_shape=[
            jax.ShapeDtypeStruct((_N_NODES, _D), jnp.float32),
            jax.ShapeDtypeStruct((_N_NODES, _D), jnp.float32),
        ],
    )(x, W_es, W_ed)


def _proj_edges_body(e_ref, wee_ref, g_ref, weg_ref, be_ref, eidx_ref,
                     pe_ref, pg_scr):
    @pl.when(pl.program_id(0) == 0)
    def _():
        pg_scr[...] = (jnp.dot(g_ref[...], weg_ref[...],
                               preferred_element_type=jnp.float32)
                       + be_ref[...])

    eidx = eidx_ref[0, 0, :]
    oh = (eidx[:, None] == lax.broadcasted_iota(jnp.int32, (eidx.shape[0], _NG), 1)
          ).astype(jnp.float32)
    pe_ref[...] = (jnp.dot(e_ref[...], wee_ref[...],
                           preferred_element_type=jnp.float32)
                   + jnp.dot(oh, pg_scr[...], preferred_element_type=jnp.float32))


def _proj_edges(e, W_ee, g, W_eg, b_e2d, edge_idx3d):
    blk = 2000
    grid = _N_EDGES // blk
    return pl.pallas_call(
        _proj_edges_body,
        grid=(grid,),
        in_specs=[
            pl.BlockSpec((blk, 16), lambda i: (i, 0)),
            pl.BlockSpec((16, _D), lambda i: (0, 0)),
            pl.BlockSpec((_NG, 64), lambda i: (0, 0)),
            pl.BlockSpec((64, _D), lambda i: (0, 0)),
            pl.BlockSpec((1, _D), lambda i: (0, 0)),
            pl.BlockSpec((1, 1, blk), lambda i: (i, 0, 0)),
        ],
        out_specs=pl.BlockSpec((blk, _D), lambda i: (i, 0)),
        out_shape=jax.ShapeDtypeStruct((_N_EDGES, _D), jnp.float32),
        scratch_shapes=[pltpu.VMEM((_NG, _D), jnp.float32)],
    )(e, W_ee, g, W_eg, b_e2d, edge_idx3d)


def _sc_edge_body(psrc, pdst, pe, srcidx, dstidx, gidx,
                  eattr, agg0, agg1, eagg0, eagg1,
                  src_v0, src_v1, dst_v0, dst_v1, gid_v0, gid_v1,
                  pe_v0, pe_v1, s_v0, s_v1, d_v0, d_v1, o_v0, o_v1,
                  agg_sh, eagg_sh,
                  sem_in0, sem_in1, sem_g0, sem_g1):
    cid = lax.axis_index("c")
    sid = lax.axis_index("s")
    wid = cid * 16 + sid
    src_v = (src_v0, src_v1)
    dst_v = (dst_v0, dst_v1)
    gid_v = (gid_v0, gid_v1)
    pe_v = (pe_v0, pe_v1)
    s_v = (s_v0, s_v1)
    d_v = (d_v0, d_v1)
    o_v = (o_v0, o_v1)
    sem_in = (sem_in0, sem_in1)
    sem_g = (sem_g0, sem_g1)

    def _zero_row(i, _):
        for c in range(_D // 16):
            o_v0[i, pl.ds(c * 16, 16)] = jnp.zeros((16,), jnp.float32)
        return 0
    lax.fori_loop(0, _CH, _zero_row, 0)
    nz = _STRIPE // _CH
    rem = _STRIPE - nz * _CH
    for j in range(nz):
        pltpu.sync_copy(o_v0.at[pl.ds(0, _CH)],
                        agg_sh.at[pl.ds(sid * _STRIPE + j * _CH, _CH)])
    if rem:
        pltpu.sync_copy(o_v0.at[pl.ds(0, rem)],
                        agg_sh.at[pl.ds(sid * _STRIPE + nz * _CH, rem)])

    @pl.when(sid == 0)
    def _():
        pltpu.sync_copy(o_v0.at[pl.ds(0, _STRIPE_TAIL)],
                        agg_sh.at[pl.ds(16 * _STRIPE, _STRIPE_TAIL)])
        pltpu.sync_copy(o_v0.at[pl.ds(0, _NG)], eagg_sh)

    plsc.subcore_barrier()

    start = wid * _CH_PER_W
    N = _CH_PER_W

    def _issue_in(t, b):
        off = (start + t) * _CH
        pltpu.async_copy(srcidx.at[pl.ds(off, _CH)], src_v[b], sem_in[b])
        pltpu.async_copy(dstidx.at[pl.ds(off, _CH)], dst_v[b], sem_in[b])
        pltpu.async_copy(gidx.at[pl.ds(off, _CH)], gid_v[b], sem_in[b])
        pltpu.async_copy(pe.at[pl.ds(off, _CH)], pe_v[b], sem_in[b])

    def _wait_in(b):
        pltpu.make_async_copy(srcidx.at[pl.ds(0, _CH)], src_v[b], sem_in[b]).wait()
        pltpu.make_async_copy(dstidx.at[pl.ds(0, _CH)], dst_v[b], sem_in[b]).wait()
        pltpu.make_async_copy(gidx.at[pl.ds(0, _CH)], gid_v[b], sem_in[b]).wait()
        pltpu.make_async_copy(pe.at[pl.ds(0, _CH)], pe_v[b], sem_in[b]).wait()

    def _issue_gather(b):
        pltpu.async_copy(psrc.at[src_v[b]], s_v[b], sem_g[b])
        pltpu.async_copy(pdst.at[dst_v[b]], d_v[b], sem_g[b])

    def _wait_gather(b):
        pltpu.make_async_copy(psrc.at[src_v[b]], s_v[b], sem_g[b]).wait()
        pltpu.make_async_copy(pdst.at[dst_v[b]], d_v[b], sem_g[b]).wait()

    def _compute(b):
        def _row(i, _):
            for c in range(_D // 16):
                sl = pl.ds(c * 16, 16)
                v = pe_v[b][i, sl] + s_v[b][i, sl] + d_v[b][i, sl]
                o_v[b][i, sl] = jnp.maximum(v, 0.0)
            return 0
        lax.fori_loop(0, _CH, _row, 0)

    def _sync_out(t, b):
        off = (start + t) * _CH
        pltpu.sync_copy(o_v[b], eattr.at[pl.ds(off, _CH)])
        pltpu.sync_copy(o_v[b], agg_sh.at[dst_v[b]], add=True)
        pltpu.sync_copy(o_v[b], eagg_sh.at[gid_v[b]], add=True)

    def _step(t, b, has_next=True, has_next2=True):
        if has_next:
            _wait_in(1 - b)
        _wait_gather(b)
        if has_next:
            _issue_gather(1 - b)
        _compute(b)
        _sync_out(t, b)
        if has_next2:
            _issue_in(t + 2, b)

    _issue_in(0, 0)
    _wait_in(0)
    _issue_gather(0)
    _issue_in(1, 1)

    def _pair(g, _):
        t0 = 2 * g
        _step(t0, 0)
        _step(t0 + 1, 1)
        return 0

    lax.fori_loop(0, (N - 2) // 2, _pair, 0)

    _step(N - 2, 0, has_next=True, has_next2=False)
    _step(N - 1, 1, has_next=False, has_next2=False)
    plsc.subcore_barrier()

    rows = pl.ds(sid * _STRIPE, _STRIPE)
    tail = pl.ds(16 * _STRIPE, _STRIPE_TAIL)

    @pl.when(cid == 0)
    def _():
        pltpu.sync_copy(agg_sh.at[rows], agg0.at[rows])

    @pl.when(cid == 1)
    def _():
        pltpu.sync_copy(agg_sh.at[rows], agg1.at[rows])

    @pl.when((cid == 0) & (sid == 0))
    def _():
        pltpu.sync_copy(agg_sh.at[tail], agg0.at[tail])
        pltpu.sync_copy(eagg_sh, eagg0)

    @pl.when((cid == 1) & (sid == 0))
    def _():
        pltpu.sync_copy(agg_sh.at[tail], agg1.at[tail])
        pltpu.sync_copy(eagg_sh, eagg1)


def _sc_edge(P_src, P_dst, P_e, src, dst, edge_idx):
    mesh = plsc.VectorSubcoreMesh(core_axis_name="c", subcore_axis_name="s")
    f = pl.kernel(
        _sc_edge_body,
        mesh=mesh,
        out_type=[
            jax.ShapeDtypeStruct((_N_EDGES, _D), jnp.float32),
            jax.ShapeDtypeStruct((_N_NODES, _D), jnp.float32),
            jax.ShapeDtypeStruct((_N_NODES, _D), jnp.float32),
            jax.ShapeDtypeStruct((_NG, _D), jnp.float32),
            jax.ShapeDtypeStruct((_NG, _D), jnp.float32),
        ],
        scratch_types=(
            [pltpu.VMEM((_CH,), jnp.int32)] * 6
            + [pltpu.VMEM((_CH, _D), jnp.float32)] * 8
            + [
                pltpu.VMEM_SHARED((_N_NODES, _D), jnp.float32),
                pltpu.VMEM_SHARED((_NG, _D), jnp.float32),
                pltpu.SemaphoreType.DMA,
                pltpu.SemaphoreType.DMA,
                pltpu.SemaphoreType.DMA,
                pltpu.SemaphoreType.DMA,
            ]
        ),
    )
    return f(P_src, P_dst, P_e, src, dst, edge_idx)


def _node_body(x_ref, a0_ref, a1_ref, nidx_ref, wn1_ref, wn2_ref,
               g_ref, wn3_ref, bn_ref, e0_ref, e1_ref,
               wg1_ref, wg2_ref, wg3_ref, bg_ref,
               na_ref, ga_ref, gn_scr, gacc_scr):
    i = pl.program_id(0)
    nsteps = pl.num_programs(0)

    @pl.when(i == 0)
    def _():
        gn_scr[...] = (jnp.dot(g_ref[...], wn3_ref[...],
                               preferred_element_type=jnp.float32)
                       + bn_ref[...])
        gacc_scr[...] = jnp.zeros_like(gacc_scr)

    nidx = nidx_ref[0, 0, :]
    nb = nidx.shape[0]
    oh = (nidx[:, None] == lax.broadcasted_iota(jnp.int32, (nb, _NG), 1)
          ).astype(jnp.float32)
    oht = (nidx[None, :] == lax.broadcasted_iota(jnp.int32, (_NG, nb), 0)
           ).astype(jnp.float32)
    agg = a0_ref[...] + a1_ref[...]
    na = (jnp.dot(x_ref[...], wn1_ref[...], preferred_element_type=jnp.float32)
          + jnp.dot(agg, wn2_ref[...], preferred_element_type=jnp.float32)
          + jnp.dot(oh, gn_scr[...], preferred_element_type=jnp.float32))
    na = jnp.maximum(na, 0.0)
    na_ref[...] = na
    gacc_scr[...] += jnp.dot(oht, na, preferred_element_type=jnp.float32)

    @pl.when(i == nsteps - 1)
    def _():
        eagg = e0_ref[...] + e1_ref[...]
        ga_ref[...] = (
            jnp.dot(g_ref[...], wg1_ref[...], preferred_element_type=jnp.float32)
            + jnp.dot(gacc_scr[...], wg2_ref[...], preferred_element_type=jnp.float32)
            + jnp.dot(eagg, wg3_ref[...], preferred_element_type=jnp.float32)
            + bg_ref[...])


def _node_global(x, agg0, agg1, nidx3d, W_n1, W_n2, g, W_n3, b_n2d,
                 eagg0, eagg1, W_g1, W_g2, W_g3, b_g2d):
    blk = 1000
    grid = _N_NODES // blk
    return pl.pallas_call(
        _node_body,
        grid=(grid,),
        in_specs=[
            pl.BlockSpec((blk, _D), lambda i: (i, 0)),
            pl.BlockSpec((blk, _D), lambda i: (i, 0)),
            pl.BlockSpec((blk, _D), lambda i: (i, 0)),
            pl.BlockSpec((1, 1, blk), lambda i: (i, 0, 0)),
            pl.BlockSpec((_D, _D), lambda i: (0, 0)),
            pl.BlockSpec((_D, _D), lambda i: (0, 0)),
            pl.BlockSpec((_NG, 64), lambda i: (0, 0)),
            pl.BlockSpec((64, _D), lambda i: (0, 0)),
            pl.BlockSpec((1, _D), lambda i: (0, 0)),
            pl.BlockSpec((_NG, _D), lambda i: (0, 0)),
            pl.BlockSpec((_NG, _D), lambda i: (0, 0)),
            pl.BlockSpec((64, 64), lambda i: (0, 0)),
            pl.BlockSpec((_D, 64), lambda i: (0, 0)),
            pl.BlockSpec((_D, 64), lambda i: (0, 0)),
            pl.BlockSpec((1, 64), lambda i: (0, 0)),
        ],
        out_specs=[
            pl.BlockSpec((blk, _D), lambda i: (i, 0)),
            pl.BlockSpec((_NG, 64), lambda i: (0, 0)),
        ],
        out_shape=[
            jax.ShapeDtypeStruct((_N_NODES, _D), jnp.float32),
            jax.ShapeDtypeStruct((_NG, 64), jnp.float32),
        ],
        scratch_shapes=[
            pltpu.VMEM((_NG, _D), jnp.float32),
            pltpu.VMEM((_NG, _D), jnp.float32),
        ],
    )(x, agg0, agg1, nidx3d, W_n1, W_n2, g, W_n3, b_n2d,
      eagg0, eagg1, W_g1, W_g2, W_g3, b_g2d)


@jax.jit
def kernel(x, e, g, W_e, b_e, W_n, b_n, W_g, b_g, edges, node_idx, edge_idx):
    src = edges[0].astype(jnp.int32)
    dst = edges[1].astype(jnp.int32)
    edge_idx = edge_idx.astype(jnp.int32)
    node_idx = node_idx.astype(jnp.int32)

    W_ee, W_es, W_ed, W_eg = W_e[:16], W_e[16:144], W_e[144:272], W_e[272:336]
    W_n1, W_n2, W_n3 = W_n[:128], W_n[128:256], W_n[256:320]
    W_g1, W_g2, W_g3 = W_g[:64], W_g[64:192], W_g[192:320]

    P_src, P_dst = _proj_nodes(x, W_es, W_ed)
    P_e = _proj_edges(e, W_ee, g, W_eg, b_e.reshape(1, _D),
                      edge_idx.reshape(_N_EDGES // 2000, 1, 2000))
    edge_attr, agg0, agg1, eagg0, eagg1 = _sc_edge(P_src, P_dst, P_e,
                                                   src, dst, edge_idx)
    node_attr, global_attr = _node_global(
        x, agg0, agg1, node_idx.reshape(_N_NODES // 1000, 1, 1000),
        W_n1, W_n2, g, W_n3, b_n.reshape(1, _D),
        eagg0, eagg1, W_g1, W_g2, W_g3, b_g.reshape(1, 64))
    return (edge_attr, node_attr, global_attr)


# speedup vs baseline: 5.4268x; 1.0593x over previous
import functools
import jax
import jax.numpy as jnp
from jax import lax
from jax.experimental import pallas as pl
from jax.experimental.pallas import tpu as pltpu
from jax.experimental.pallas import tpu_sc as plsc

_N_NODES = 10000
_N_EDGES = 320000
_D = 128
_NG = 16
_CH = 40
_NCHUNKS = _N_EDGES // _CH
_NWORKERS = 32
_CH_PER_W = _NCHUNKS // _NWORKERS
_STRIPE = 624
_STRIPE_TAIL = _N_NODES - 16 * _STRIPE


def _proj_nodes_body(x_ref, wes_ref, wed_ref, ps_ref, pd_ref):
    xb = x_ref[...]
    ps_ref[...] = jnp.dot(xb, wes_ref[...], preferred_element_type=jnp.float32)
    pd_ref[...] = jnp.dot(xb, wed_ref[...], preferred_element_type=jnp.float32)


def _proj_nodes(x, W_es, W_ed):
    blk = 1000
    grid = _N_NODES // blk
    return pl.pallas_call(
        _proj_nodes_body,
        grid=(grid,),
        in_specs=[
            pl.BlockSpec((blk, _D), lambda i: (i, 0)),
            pl.BlockSpec((_D, _D), lambda i: (0, 0)),
            pl.BlockSpec((_D, _D), lambda i: (0, 0)),
        ],
        out_specs=[
            pl.BlockSpec((blk, _D), lambda i: (i, 0)),
            pl.BlockSpec((blk, _D), lambda i: (i, 0)),
        ],
        out_shape=[
            jax.ShapeDtypeStruct((_N_NODES, _D), jnp.float32),
            jax.ShapeDtypeStruct((_N_NODES, _D), jnp.float32),
        ],
    )(x, W_es, W_ed)


def _proj_edges_body(e_ref, wee_ref, g_ref, weg_ref, be_ref, eidx_ref,
                     pe_ref, pg_scr):
    @pl.when(pl.program_id(0) == 0)
    def _():
        pg_scr[...] = (jnp.dot(g_ref[...], weg_ref[...],
                               preferred_element_type=jnp.float32)
                       + be_ref[...])

    eidx = eidx_ref[0, 0, :]
    oh = (eidx[:, None] == lax.broadcasted_iota(jnp.int32, (eidx.shape[0], _NG), 1)
          ).astype(jnp.float32)
    pe_ref[...] = (jnp.dot(e_ref[...], wee_ref[...],
                           preferred_element_type=jnp.float32)
                   + jnp.dot(oh, pg_scr[...], preferred_element_type=jnp.float32))


def _proj_edges(e, W_ee, g, W_eg, b_e2d, edge_idx3d):
    blk = 2000
    grid = _N_EDGES // blk
    return pl.pallas_call(
        _proj_edges_body,
        grid=(grid,),
        in_specs=[
            pl.BlockSpec((blk, 16), lambda i: (i, 0)),
            pl.BlockSpec((16, _D), lambda i: (0, 0)),
            pl.BlockSpec((_NG, 64), lambda i: (0, 0)),
            pl.BlockSpec((64, _D), lambda i: (0, 0)),
            pl.BlockSpec((1, _D), lambda i: (0, 0)),
            pl.BlockSpec((1, 1, blk), lambda i: (i, 0, 0)),
        ],
        out_specs=pl.BlockSpec((blk, _D), lambda i: (i, 0)),
        out_shape=jax.ShapeDtypeStruct((_N_EDGES, _D), jnp.float32),
        scratch_shapes=[pltpu.VMEM((_NG, _D), jnp.float32)],
    )(e, W_ee, g, W_eg, b_e2d, edge_idx3d)


def _sc_edge_body(psrc, pdst, pe, srcidx, dstidx, gidx,
                  eattr, agg0, agg1, eagg0, eagg1,
                  src_v0, src_v1, dst_v0, dst_v1, gid_v0, gid_v1,
                  pe_v0, pe_v1, s_v0, s_v1, d_v0, d_v1, o_v0, o_v1,
                  agg_sh, eagg_sh,
                  sem_in0, sem_in1, sem_g0, sem_g1, sem_w0, sem_w1, sem_s):
    cid = lax.axis_index("c")
    sid = lax.axis_index("s")
    wid = cid * 16 + sid
    src_v = (src_v0, src_v1)
    dst_v = (dst_v0, dst_v1)
    gid_v = (gid_v0, gid_v1)
    pe_v = (pe_v0, pe_v1)
    s_v = (s_v0, s_v1)
    d_v = (d_v0, d_v1)
    o_v = (o_v0, o_v1)
    sem_in = (sem_in0, sem_in1)
    sem_g = (sem_g0, sem_g1)
    sem_w = (sem_w0, sem_w1)

    def _zero_row(i, _):
        for c in range(_D // 16):
            o_v0[i, pl.ds(c * 16, 16)] = jnp.zeros((16,), jnp.float32)
        return 0
    lax.fori_loop(0, _CH, _zero_row, 0)
    nz = _STRIPE // _CH
    rem = _STRIPE - nz * _CH
    for j in range(nz):
        pltpu.sync_copy(o_v0.at[pl.ds(0, _CH)],
                        agg_sh.at[pl.ds(sid * _STRIPE + j * _CH, _CH)])
    if rem:
        pltpu.sync_copy(o_v0.at[pl.ds(0, rem)],
                        agg_sh.at[pl.ds(sid * _STRIPE + nz * _CH, rem)])

    @pl.when(sid == 0)
    def _():
        pltpu.sync_copy(o_v0.at[pl.ds(0, _STRIPE_TAIL)],
                        agg_sh.at[pl.ds(16 * _STRIPE, _STRIPE_TAIL)])
        pltpu.sync_copy(o_v0.at[pl.ds(0, _NG)], eagg_sh)

    plsc.subcore_barrier()

    start = wid * _CH_PER_W
    N = _CH_PER_W

    def _issue_in(t, b):
        off = (start + t) * _CH
        pltpu.async_copy(srcidx.at[pl.ds(off, _CH)], src_v[b], sem_in[b])
        pltpu.async_copy(dstidx.at[pl.ds(off, _CH)], dst_v[b], sem_in[b])
        pltpu.async_copy(gidx.at[pl.ds(off, _CH)], gid_v[b], sem_in[b])
        pltpu.async_copy(pe.at[pl.ds(off, _CH)], pe_v[b], sem_in[b])

    def _wait_in(b):
        pltpu.make_async_copy(srcidx.at[pl.ds(0, _CH)], src_v[b], sem_in[b]).wait()
        pltpu.make_async_copy(dstidx.at[pl.ds(0, _CH)], dst_v[b], sem_in[b]).wait()
        pltpu.make_async_copy(gidx.at[pl.ds(0, _CH)], gid_v[b], sem_in[b]).wait()
        pltpu.make_async_copy(pe.at[pl.ds(0, _CH)], pe_v[b], sem_in[b]).wait()

    def _issue_gather(b):
        pltpu.async_copy(psrc.at[src_v[b]], s_v[b], sem_g[b])
        pltpu.async_copy(pdst.at[dst_v[b]], d_v[b], sem_g[b])

    def _wait_gather(b):
        pltpu.make_async_copy(psrc.at[src_v[b]], s_v[b], sem_g[b]).wait()
        pltpu.make_async_copy(pdst.at[dst_v[b]], d_v[b], sem_g[b]).wait()

    def _compute(b):
        def _row(i, _):
            for c in range(_D // 16):
                sl = pl.ds(c * 16, 16)
                v = pe_v[b][i, sl] + s_v[b][i, sl] + d_v[b][i, sl]
                o_v[b][i, sl] = jnp.maximum(v, 0.0)
            return 0
        lax.fori_loop(0, _CH, _row, 0)

    def _issue_w(t, b):
        off = (start + t) * _CH
        pltpu.async_copy(o_v[b], eattr.at[pl.ds(off, _CH)], sem_w[b])

    def _wait_w(b):
        pltpu.make_async_copy(o_v[b], eattr.at[pl.ds(0, _CH)], sem_w[b]).wait()

    def _scatter_adds(b):
        c1 = pltpu.async_copy(o_v[b], agg_sh.at[dst_v[b]], sem_s, add=True)
        c2 = pltpu.async_copy(o_v[b], eagg_sh.at[gid_v[b]], sem_s, add=True)
        c1.wait()
        c2.wait()

    def _step(t, b, first2=False, has_next=True, has_next2=True):
        if has_next:
            _wait_in(1 - b)
        _wait_gather(b)
        if has_next:
            _issue_gather(1 - b)
        if not first2:
            _wait_w(b)
        _compute(b)
        _issue_w(t, b)
        _scatter_adds(b)
        if has_next2:
            _issue_in(t + 2, b)

    _issue_in(0, 0)
    _wait_in(0)
    _issue_gather(0)
    _issue_in(1, 1)
    _step(0, 0, first2=True)
    _step(1, 1, first2=True)

    def _pair(g, _):
        t0 = 2 * g
        _step(t0, 0)
        _step(t0 + 1, 1)
        return 0

    lax.fori_loop(1, (N - 2) // 2, _pair, 0)

    _step(N - 2, 0, has_next=True, has_next2=False)
    _step(N - 1, 1, has_next=False, has_next2=False)
    _wait_w(0)
    _wait_w(1)
    plsc.subcore_barrier()

    rows = pl.ds(sid * _STRIPE, _STRIPE)
    tail = pl.ds(16 * _STRIPE, _STRIPE_TAIL)

    @pl.when(cid == 0)
    def _():
        pltpu.sync_copy(agg_sh.at[rows], agg0.at[rows])

    @pl.when(cid == 1)
    def _():
        pltpu.sync_copy(agg_sh.at[rows], agg1.at[rows])

    @pl.when((cid == 0) & (sid == 0))
    def _():
        pltpu.sync_copy(agg_sh.at[tail], agg0.at[tail])
        pltpu.sync_copy(eagg_sh, eagg0)

    @pl.when((cid == 1) & (sid == 0))
    def _():
        pltpu.sync_copy(agg_sh.at[tail], agg1.at[tail])
        pltpu.sync_copy(eagg_sh, eagg1)


def _sc_edge(P_src, P_dst, P_e, src, dst, edge_idx):
    mesh = plsc.VectorSubcoreMesh(core_axis_name="c", subcore_axis_name="s")
    f = pl.kernel(
        _sc_edge_body,
        mesh=mesh,
        out_type=[
            jax.ShapeDtypeStruct((_N_EDGES, _D), jnp.float32),
            jax.ShapeDtypeStruct((_N_NODES, _D), jnp.float32),
            jax.ShapeDtypeStruct((_N_NODES, _D), jnp.float32),
            jax.ShapeDtypeStruct((_NG, _D), jnp.float32),
            jax.ShapeDtypeStruct((_NG, _D), jnp.float32),
        ],
        scratch_types=(
            [pltpu.VMEM((_CH,), jnp.int32)] * 6
            + [pltpu.VMEM((_CH, _D), jnp.float32)] * 8
            + [
                pltpu.VMEM_SHARED((_N_NODES, _D), jnp.float32),
                pltpu.VMEM_SHARED((_NG, _D), jnp.float32),
                pltpu.SemaphoreType.DMA,
                pltpu.SemaphoreType.DMA,
                pltpu.SemaphoreType.DMA,
                pltpu.SemaphoreType.DMA,
                pltpu.SemaphoreType.DMA,
                pltpu.SemaphoreType.DMA,
                pltpu.SemaphoreType.DMA,
            ]
        ),
    )
    return f(P_src, P_dst, P_e, src, dst, edge_idx)


def _node_body(x_ref, a0_ref, a1_ref, nidx_ref, wn1_ref, wn2_ref,
               g_ref, wn3_ref, bn_ref, e0_ref, e1_ref,
               wg1_ref, wg2_ref, wg3_ref, bg_ref,
               na_ref, ga_ref, gn_scr, gacc_scr):
    i = pl.program_id(0)
    nsteps = pl.num_programs(0)

    @pl.when(i == 0)
    def _():
        gn_scr[...] = (jnp.dot(g_ref[...], wn3_ref[...],
                               preferred_element_type=jnp.float32)
                       + bn_ref[...])
        gacc_scr[...] = jnp.zeros_like(gacc_scr)

    nidx = nidx_ref[0, 0, :]
    nb = nidx.shape[0]
    oh = (nidx[:, None] == lax.broadcasted_iota(jnp.int32, (nb, _NG), 1)
          ).astype(jnp.float32)
    oht = (nidx[None, :] == lax.broadcasted_iota(jnp.int32, (_NG, nb), 0)
           ).astype(jnp.float32)
    agg = a0_ref[...] + a1_ref[...]
    na = (jnp.dot(x_ref[...], wn1_ref[...], preferred_element_type=jnp.float32)
          + jnp.dot(agg, wn2_ref[...], preferred_element_type=jnp.float32)
          + jnp.dot(oh, gn_scr[...], preferred_element_type=jnp.float32))
    na = jnp.maximum(na, 0.0)
    na_ref[...] = na
    gacc_scr[...] += jnp.dot(oht, na, preferred_element_type=jnp.float32)

    @pl.when(i == nsteps - 1)
    def _():
        eagg = e0_ref[...] + e1_ref[...]
        ga_ref[...] = (
            jnp.dot(g_ref[...], wg1_ref[...], preferred_element_type=jnp.float32)
            + jnp.dot(gacc_scr[...], wg2_ref[...], preferred_element_type=jnp.float32)
            + jnp.dot(eagg, wg3_ref[...], preferred_element_type=jnp.float32)
            + bg_ref[...])


def _node_global(x, agg0, agg1, nidx3d, W_n1, W_n2, g, W_n3, b_n2d,
                 eagg0, eagg1, W_g1, W_g2, W_g3, b_g2d):
    blk = 1000
    grid = _N_NODES // blk
    return pl.pallas_call(
        _node_body,
        grid=(grid,),
        in_specs=[
            pl.BlockSpec((blk, _D), lambda i: (i, 0)),
            pl.BlockSpec((blk, _D), lambda i: (i, 0)),
            pl.BlockSpec((blk, _D), lambda i: (i, 0)),
            pl.BlockSpec((1, 1, blk), lambda i: (i, 0, 0)),
            pl.BlockSpec((_D, _D), lambda i: (0, 0)),
            pl.BlockSpec((_D, _D), lambda i: (0, 0)),
            pl.BlockSpec((_NG, 64), lambda i: (0, 0)),
            pl.BlockSpec((64, _D), lambda i: (0, 0)),
            pl.BlockSpec((1, _D), lambda i: (0, 0)),
            pl.BlockSpec((_NG, _D), lambda i: (0, 0)),
            pl.BlockSpec((_NG, _D), lambda i: (0, 0)),
            pl.BlockSpec((64, 64), lambda i: (0, 0)),
            pl.BlockSpec((_D, 64), lambda i: (0, 0)),
            pl.BlockSpec((_D, 64), lambda i: (0, 0)),
            pl.BlockSpec((1, 64), lambda i: (0, 0)),
        ],
        out_specs=[
            pl.BlockSpec((blk, _D), lambda i: (i, 0)),
            pl.BlockSpec((_NG, 64), lambda i: (0, 0)),
        ],
        out_shape=[
            jax.ShapeDtypeStruct((_N_NODES, _D), jnp.float32),
            jax.ShapeDtypeStruct((_NG, 64), jnp.float32),
        ],
        scratch_shapes=[
            pltpu.VMEM((_NG, _D), jnp.float32),
            pltpu.VMEM((_NG, _D), jnp.float32),
        ],
    )(x, agg0, agg1, nidx3d, W_n1, W_n2, g, W_n3, b_n2d,
      eagg0, eagg1, W_g1, W_g2, W_g3, b_g2d)


@jax.jit
def kernel(x, e, g, W_e, b_e, W_n, b_n, W_g, b_g, edges, node_idx, edge_idx):
    src = edges[0].astype(jnp.int32)
    dst = edges[1].astype(jnp.int32)
    edge_idx = edge_idx.astype(jnp.int32)
    node_idx = node_idx.astype(jnp.int32)

    W_ee, W_es, W_ed, W_eg = W_e[:16], W_e[16:144], W_e[144:272], W_e[272:336]
    W_n1, W_n2, W_n3 = W_n[:128], W_n[128:256], W_n[256:320]
    W_g1, W_g2, W_g3 = W_g[:64], W_g[64:192], W_g[192:320]

    P_src, P_dst = _proj_nodes(x, W_es, W_ed)
    P_e = _proj_edges(e, W_ee, g, W_eg, b_e.reshape(1, _D),
                      edge_idx.reshape(_N_EDGES // 2000, 1, 2000))
    edge_attr, agg0, agg1, eagg0, eagg1 = _sc_edge(P_src, P_dst, P_e,
                                                   src, dst, edge_idx)
    node_attr, global_attr = _node_global(
        x, agg0, agg1, node_idx.reshape(_N_NODES // 1000, 1, 1000),
        W_n1, W_n2, g, W_n3, b_n.reshape(1, _D),
        eagg0, eagg1, W_g1, W_g2, W_g3, b_g.reshape(1, 64))
    return (edge_attr, node_attr, global_attr)


# speedup vs baseline: 6.1236x; 1.1284x over previous
import functools
import jax
import jax.numpy as jnp
from jax import lax
from jax.experimental import pallas as pl
from jax.experimental.pallas import tpu as pltpu
from jax.experimental.pallas import tpu_sc as plsc

_N_NODES = 10000
_N_EDGES = 320000
_D = 128
_NG = 16
_CH = 40
_NCHUNKS = _N_EDGES // _CH
_NWORKERS = 32
_CH_PER_W = _NCHUNKS // _NWORKERS
_STRIPE = 624
_STRIPE_TAIL = _N_NODES - 16 * _STRIPE


def _proj_nodes_body(x_ref, wes_ref, wed_ref, ps_ref, pd_ref):
    xb = x_ref[...]
    ps_ref[...] = jnp.dot(xb, wes_ref[...], preferred_element_type=jnp.float32)
    pd_ref[...] = jnp.dot(xb, wed_ref[...], preferred_element_type=jnp.float32)


def _proj_nodes(x, W_es, W_ed):
    blk = 1000
    grid = _N_NODES // blk
    return pl.pallas_call(
        _proj_nodes_body,
        grid=(grid,),
        in_specs=[
            pl.BlockSpec((blk, _D), lambda i: (i, 0)),
            pl.BlockSpec((_D, _D), lambda i: (0, 0)),
            pl.BlockSpec((_D, _D), lambda i: (0, 0)),
        ],
        out_specs=[
            pl.BlockSpec((blk, _D), lambda i: (i, 0)),
            pl.BlockSpec((blk, _D), lambda i: (i, 0)),
        ],
        out_shape=[
            jax.ShapeDtypeStruct((_N_NODES, _D), jnp.float32),
            jax.ShapeDtypeStruct((_N_NODES, _D), jnp.float32),
        ],
    )(x, W_es, W_ed)


def _proj_edges_body(e_ref, wee_ref, g_ref, weg_ref, be_ref, eidx_ref,
                     pe_ref, pg_scr):
    @pl.when(pl.program_id(0) == 0)
    def _():
        pg_scr[...] = (jnp.dot(g_ref[...], weg_ref[...],
                               preferred_element_type=jnp.float32)
                       + be_ref[...])

    eidx = eidx_ref[0, 0, :]
    oh = (eidx[:, None] == lax.broadcasted_iota(jnp.int32, (eidx.shape[0], _NG), 1)
          ).astype(jnp.float32)
    pe_ref[...] = (jnp.dot(e_ref[...], wee_ref[...],
                           preferred_element_type=jnp.float32)
                   + jnp.dot(oh, pg_scr[...], preferred_element_type=jnp.float32))


def _proj_edges(e, W_ee, g, W_eg, b_e2d, edge_idx3d):
    blk = 2000
    grid = _N_EDGES // blk
    return pl.pallas_call(
        _proj_edges_body,
        grid=(grid,),
        in_specs=[
            pl.BlockSpec((blk, 16), lambda i: (i, 0)),
            pl.BlockSpec((16, _D), lambda i: (0, 0)),
            pl.BlockSpec((_NG, 64), lambda i: (0, 0)),
            pl.BlockSpec((64, _D), lambda i: (0, 0)),
            pl.BlockSpec((1, _D), lambda i: (0, 0)),
            pl.BlockSpec((1, 1, blk), lambda i: (i, 0, 0)),
        ],
        out_specs=pl.BlockSpec((blk, _D), lambda i: (i, 0)),
        out_shape=jax.ShapeDtypeStruct((_N_EDGES, _D), jnp.float32),
        scratch_shapes=[pltpu.VMEM((_NG, _D), jnp.float32)],
    )(e, W_ee, g, W_eg, b_e2d, edge_idx3d)


def _sc_edge_body(psrc, pdst, pe, srcidx, dstidx, gidx,
                  eattr, agg0, agg1, eagg0, eagg1,
                  src_v0, src_v1, src_v2, src_v3,
                  dst_v0, dst_v1, dst_v2, dst_v3,
                  gid_v0, gid_v1, gid_v2, gid_v3,
                  pe_v0, pe_v1, s_v0, s_v1, d_v0, d_v1, o_v0, o_v1,
                  agg_sh, eagg_sh,
                  sem_in0, sem_in1, sem_g0, sem_g1, sem_w0, sem_w1,
                  sem_s0, sem_s1):
    cid = lax.axis_index("c")
    sid = lax.axis_index("s")
    wid = cid * 16 + sid
    src_v = (src_v0, src_v1, src_v2, src_v3)
    dst_v = (dst_v0, dst_v1, dst_v2, dst_v3)
    gid_v = (gid_v0, gid_v1, gid_v2, gid_v3)
    pe_v = (pe_v0, pe_v1)
    s_v = (s_v0, s_v1)
    d_v = (d_v0, d_v1)
    o_v = (o_v0, o_v1)
    sem_in = (sem_in0, sem_in1)
    sem_g = (sem_g0, sem_g1)
    sem_w = (sem_w0, sem_w1)
    sem_s = (sem_s0, sem_s1)

    def _zero_row(i, _):
        for c in range(_D // 16):
            o_v0[i, pl.ds(c * 16, 16)] = jnp.zeros((16,), jnp.float32)
        return 0
    lax.fori_loop(0, _CH, _zero_row, 0)
    nz = _STRIPE // _CH
    rem = _STRIPE - nz * _CH
    for j in range(nz):
        pltpu.sync_copy(o_v0.at[pl.ds(0, _CH)],
                        agg_sh.at[pl.ds(sid * _STRIPE + j * _CH, _CH)])
    if rem:
        pltpu.sync_copy(o_v0.at[pl.ds(0, rem)],
                        agg_sh.at[pl.ds(sid * _STRIPE + nz * _CH, rem)])

    @pl.when(sid == 0)
    def _():
        pltpu.sync_copy(o_v0.at[pl.ds(0, _STRIPE_TAIL)],
                        agg_sh.at[pl.ds(16 * _STRIPE, _STRIPE_TAIL)])
        pltpu.sync_copy(o_v0.at[pl.ds(0, _NG)], eagg_sh)

    plsc.subcore_barrier()

    start = wid * _CH_PER_W
    N = _CH_PER_W

    def _issue_in(t, s2, s4):
        off = (start + t) * _CH
        pltpu.async_copy(srcidx.at[pl.ds(off, _CH)], src_v[s4], sem_in[s2])
        pltpu.async_copy(dstidx.at[pl.ds(off, _CH)], dst_v[s4], sem_in[s2])
        pltpu.async_copy(gidx.at[pl.ds(off, _CH)], gid_v[s4], sem_in[s2])
        pltpu.async_copy(pe.at[pl.ds(off, _CH)], pe_v[s2], sem_in[s2])

    def _wait_in(s2, s4):
        pltpu.make_async_copy(srcidx.at[pl.ds(0, _CH)], src_v[s4], sem_in[s2]).wait()
        pltpu.make_async_copy(dstidx.at[pl.ds(0, _CH)], dst_v[s4], sem_in[s2]).wait()
        pltpu.make_async_copy(gidx.at[pl.ds(0, _CH)], gid_v[s4], sem_in[s2]).wait()
        pltpu.make_async_copy(pe.at[pl.ds(0, _CH)], pe_v[s2], sem_in[s2]).wait()

    def _issue_gather(s2, s4):
        pltpu.async_copy(psrc.at[src_v[s4]], s_v[s2], sem_g[s2])
        pltpu.async_copy(pdst.at[dst_v[s4]], d_v[s2], sem_g[s2])

    def _wait_gather(s2, s4):
        pltpu.make_async_copy(psrc.at[src_v[s4]], s_v[s2], sem_g[s2]).wait()
        pltpu.make_async_copy(pdst.at[dst_v[s4]], d_v[s2], sem_g[s2]).wait()

    def _compute(s2):
        def _row(i, _):
            for c in range(_D // 16):
                sl = pl.ds(c * 16, 16)
                v = pe_v[s2][i, sl] + s_v[s2][i, sl] + d_v[s2][i, sl]
                o_v[s2][i, sl] = jnp.maximum(v, 0.0)
            return 0
        lax.fori_loop(0, _CH, _row, 0)

    def _issue_w(t, s2):
        off = (start + t) * _CH
        pltpu.async_copy(o_v[s2], eattr.at[pl.ds(off, _CH)], sem_w[s2])

    def _wait_w(s2):
        pltpu.make_async_copy(o_v[s2], eattr.at[pl.ds(0, _CH)], sem_w[s2]).wait()

    def _issue_s(s2, s4):
        pltpu.async_copy(o_v[s2], agg_sh.at[dst_v[s4]], sem_s[s2], add=True)
        pltpu.async_copy(o_v[s2], eagg_sh.at[gid_v[s4]], sem_s[s2], add=True)

    def _wait_s(s2, s4):
        pltpu.make_async_copy(o_v[s2], agg_sh.at[dst_v[s4]], sem_s[s2]).wait()
        pltpu.make_async_copy(o_v[s2], eagg_sh.at[gid_v[s4]], sem_s[s2]).wait()

    def _step(t, s2, s4, first2=False, has_next=True, has_next2=True):
        if has_next:
            _wait_in((s2 + 1) % 2, (s4 + 1) % 4)
        _wait_gather(s2, s4)
        if has_next:
            _issue_gather((s2 + 1) % 2, (s4 + 1) % 4)
        if not first2:
            _wait_w(s2)
            _wait_s(s2, (s4 + 2) % 4)
        _compute(s2)
        _issue_w(t, s2)
        _issue_s(s2, s4)
        if has_next2:
            _issue_in(t + 2, s2, (s4 + 2) % 4)

    _issue_in(0, 0, 0)
    _wait_in(0, 0)
    _issue_gather(0, 0)
    _issue_in(1, 1, 1)
    for t in range(4):
        _step(t, t % 2, t % 4, first2=(t < 2))

    def _quad(g, _):
        t0 = 4 * g + 4
        for q in range(4):
            _step(t0 + q, q % 2, q % 4)
        return 0

    lax.fori_loop(0, (N - 6) // 4, _quad, 0)

    _step(N - 2, (N - 2) % 2, (N - 2) % 4, has_next=True, has_next2=False)
    _step(N - 1, (N - 1) % 2, (N - 1) % 4, has_next=False, has_next2=False)
    _wait_w(0)
    _wait_s(0, (N - 2) % 4)
    _wait_w(1)
    _wait_s(1, (N - 1) % 4)
    plsc.subcore_barrier()

    rows = pl.ds(sid * _STRIPE, _STRIPE)
    tail = pl.ds(16 * _STRIPE, _STRIPE_TAIL)

    @pl.when(cid == 0)
    def _():
        pltpu.sync_copy(agg_sh.at[rows], agg0.at[rows])

    @pl.when(cid == 1)
    def _():
        pltpu.sync_copy(agg_sh.at[rows], agg1.at[rows])

    @pl.when((cid == 0) & (sid == 0))
    def _():
        pltpu.sync_copy(agg_sh.at[tail], agg0.at[tail])
        pltpu.sync_copy(eagg_sh, eagg0)

    @pl.when((cid == 1) & (sid == 0))
    def _():
        pltpu.sync_copy(agg_sh.at[tail], agg1.at[tail])
        pltpu.sync_copy(eagg_sh, eagg1)


def _sc_edge(P_src, P_dst, P_e, src, dst, edge_idx):
    mesh = plsc.VectorSubcoreMesh(core_axis_name="c", subcore_axis_name="s")
    f = pl.kernel(
        _sc_edge_body,
        mesh=mesh,
        out_type=[
            jax.ShapeDtypeStruct((_N_EDGES, _D), jnp.float32),
            jax.ShapeDtypeStruct((_N_NODES, _D), jnp.float32),
            jax.ShapeDtypeStruct((_N_NODES, _D), jnp.float32),
            jax.ShapeDtypeStruct((_NG, _D), jnp.float32),
            jax.ShapeDtypeStruct((_NG, _D), jnp.float32),
        ],
        scratch_types=(
            [pltpu.VMEM((_CH,), jnp.int32)] * 12
            + [pltpu.VMEM((_CH, _D), jnp.float32)] * 8
            + [
                pltpu.VMEM_SHARED((_N_NODES, _D), jnp.float32),
                pltpu.VMEM_SHARED((_NG, _D), jnp.float32),
            ]
            + [pltpu.SemaphoreType.DMA] * 8
        ),
    )
    return f(P_src, P_dst, P_e, src, dst, edge_idx)


def _node_body(x_ref, a0_ref, a1_ref, nidx_ref, wn1_ref, wn2_ref,
               g_ref, wn3_ref, bn_ref, e0_ref, e1_ref,
               wg1_ref, wg2_ref, wg3_ref, bg_ref,
               na_ref, ga_ref, gn_scr, gacc_scr):
    i = pl.program_id(0)
    nsteps = pl.num_programs(0)

    @pl.when(i == 0)
    def _():
        gn_scr[...] = (jnp.dot(g_ref[...], wn3_ref[...],
                               preferred_element_type=jnp.float32)
                       + bn_ref[...])
        gacc_scr[...] = jnp.zeros_like(gacc_scr)

    nidx = nidx_ref[0, 0, :]
    nb = nidx.shape[0]
    oh = (nidx[:, None] == lax.broadcasted_iota(jnp.int32, (nb, _NG), 1)
          ).astype(jnp.float32)
    oht = (nidx[None, :] == lax.broadcasted_iota(jnp.int32, (_NG, nb), 0)
           ).astype(jnp.float32)
    agg = a0_ref[...] + a1_ref[...]
    na = (jnp.dot(x_ref[...], wn1_ref[...], preferred_element_type=jnp.float32)
          + jnp.dot(agg, wn2_ref[...], preferred_element_type=jnp.float32)
          + jnp.dot(oh, gn_scr[...], preferred_element_type=jnp.float32))
    na = jnp.maximum(na, 0.0)
    na_ref[...] = na
    gacc_scr[...] += jnp.dot(oht, na, preferred_element_type=jnp.float32)

    @pl.when(i == nsteps - 1)
    def _():
        eagg = e0_ref[...] + e1_ref[...]
        ga_ref[...] = (
            jnp.dot(g_ref[...], wg1_ref[...], preferred_element_type=jnp.float32)
            + jnp.dot(gacc_scr[...], wg2_ref[...], preferred_element_type=jnp.float32)
            + jnp.dot(eagg, wg3_ref[...], preferred_element_type=jnp.float32)
            + bg_ref[...])


def _node_global(x, agg0, agg1, nidx3d, W_n1, W_n2, g, W_n3, b_n2d,
                 eagg0, eagg1, W_g1, W_g2, W_g3, b_g2d):
    blk = 1000
    grid = _N_NODES // blk
    return pl.pallas_call(
        _node_body,
        grid=(grid,),
        in_specs=[
            pl.BlockSpec((blk, _D), lambda i: (i, 0)),
            pl.BlockSpec((blk, _D), lambda i: (i, 0)),
            pl.BlockSpec((blk, _D), lambda i: (i, 0)),
            pl.BlockSpec((1, 1, blk), lambda i: (i, 0, 0)),
            pl.BlockSpec((_D, _D), lambda i: (0, 0)),
            pl.BlockSpec((_D, _D), lambda i: (0, 0)),
            pl.BlockSpec((_NG, 64), lambda i: (0, 0)),
            pl.BlockSpec((64, _D), lambda i: (0, 0)),
            pl.BlockSpec((1, _D), lambda i: (0, 0)),
            pl.BlockSpec((_NG, _D), lambda i: (0, 0)),
            pl.BlockSpec((_NG, _D), lambda i: (0, 0)),
            pl.BlockSpec((64, 64), lambda i: (0, 0)),
            pl.BlockSpec((_D, 64), lambda i: (0, 0)),
            pl.BlockSpec((_D, 64), lambda i: (0, 0)),
            pl.BlockSpec((1, 64), lambda i: (0, 0)),
        ],
        out_specs=[
            pl.BlockSpec((blk, _D), lambda i: (i, 0)),
            pl.BlockSpec((_NG, 64), lambda i: (0, 0)),
        ],
        out_shape=[
            jax.ShapeDtypeStruct((_N_NODES, _D), jnp.float32),
            jax.ShapeDtypeStruct((_NG, 64), jnp.float32),
        ],
        scratch_shapes=[
            pltpu.VMEM((_NG, _D), jnp.float32),
            pltpu.VMEM((_NG, _D), jnp.float32),
        ],
    )(x, agg0, agg1, nidx3d, W_n1, W_n2, g, W_n3, b_n2d,
      eagg0, eagg1, W_g1, W_g2, W_g3, b_g2d)


@jax.jit
def kernel(x, e, g, W_e, b_e, W_n, b_n, W_g, b_g, edges, node_idx, edge_idx):
    src = edges[0].astype(jnp.int32)
    dst = edges[1].astype(jnp.int32)
    edge_idx = edge_idx.astype(jnp.int32)
    node_idx = node_idx.astype(jnp.int32)

    W_ee, W_es, W_ed, W_eg = W_e[:16], W_e[16:144], W_e[144:272], W_e[272:336]
    W_n1, W_n2, W_n3 = W_n[:128], W_n[128:256], W_n[256:320]
    W_g1, W_g2, W_g3 = W_g[:64], W_g[64:192], W_g[192:320]

    P_src, P_dst = _proj_nodes(x, W_es, W_ed)
    P_e = _proj_edges(e, W_ee, g, W_eg, b_e.reshape(1, _D),
                      edge_idx.reshape(_N_EDGES // 2000, 1, 2000))
    edge_attr, agg0, agg1, eagg0, eagg1 = _sc_edge(P_src, P_dst, P_e,
                                                   src, dst, edge_idx)
    node_attr, global_attr = _node_global(
        x, agg0, agg1, node_idx.reshape(_N_NODES // 1000, 1, 1000),
        W_n1, W_n2, g, W_n3, b_n.reshape(1, _D),
        eagg0, eagg1, W_g1, W_g2, W_g3, b_g.reshape(1, 64))
    return (edge_attr, node_attr, global_attr)


# speedup vs baseline: 6.2047x; 1.0132x over previous
import functools
import jax
import jax.numpy as jnp
from jax import lax
from jax.experimental import pallas as pl
from jax.experimental.pallas import tpu as pltpu
from jax.experimental.pallas import tpu_sc as plsc

_N_NODES = 10000
_N_EDGES = 320000
_D = 128
_NG = 16
_CH = 40
_NCHUNKS = _N_EDGES // _CH
_NWORKERS = 32
_CH_PER_W = _NCHUNKS // _NWORKERS
_STRIPE = 624
_STRIPE_TAIL = _N_NODES - 16 * _STRIPE


def _proj_edges_body(e_ref, wee_ref, g_ref, weg_ref, be_ref, eidx_ref,
                     x_ref, wes_ref, wed_ref,
                     pe_ref, ps_ref, pd_ref, pg_scr):
    i = pl.program_id(0)

    @pl.when(i == 0)
    def _():
        pg_scr[...] = (jnp.dot(g_ref[...], weg_ref[...],
                               preferred_element_type=jnp.float32)
                       + be_ref[...])

    @pl.when(i < _N_NODES // 1000)
    def _():
        xb = x_ref[...]
        ps_ref[...] = jnp.dot(xb, wes_ref[...],
                              preferred_element_type=jnp.float32)
        pd_ref[...] = jnp.dot(xb, wed_ref[...],
                              preferred_element_type=jnp.float32)

    eidx = eidx_ref[0, 0, :]
    oh = (eidx[:, None] == lax.broadcasted_iota(jnp.int32, (eidx.shape[0], _NG), 1)
          ).astype(jnp.float32)
    pe_ref[...] = (jnp.dot(e_ref[...], wee_ref[...],
                           preferred_element_type=jnp.float32)
                   + jnp.dot(oh, pg_scr[...], preferred_element_type=jnp.float32))


def _proj_edges(e, W_ee, g, W_eg, b_e2d, edge_idx3d, x, W_es, W_ed):
    blk = 2000
    grid = _N_EDGES // blk
    nblk = 1000
    nlast = _N_NODES // nblk - 1

    def _nmap(i):
        return (jnp.minimum(i, nlast), 0)

    return pl.pallas_call(
        _proj_edges_body,
        grid=(grid,),
        in_specs=[
            pl.BlockSpec((blk, 16), lambda i: (i, 0)),
            pl.BlockSpec((16, _D), lambda i: (0, 0)),
            pl.BlockSpec((_NG, 64), lambda i: (0, 0)),
            pl.BlockSpec((64, _D), lambda i: (0, 0)),
            pl.BlockSpec((1, _D), lambda i: (0, 0)),
            pl.BlockSpec((1, 1, blk), lambda i: (i, 0, 0)),
            pl.BlockSpec((nblk, _D), _nmap),
            pl.BlockSpec((_D, _D), lambda i: (0, 0)),
            pl.BlockSpec((_D, _D), lambda i: (0, 0)),
        ],
        out_specs=[
            pl.BlockSpec((blk, _D), lambda i: (i, 0)),
            pl.BlockSpec((nblk, _D), _nmap),
            pl.BlockSpec((nblk, _D), _nmap),
        ],
        out_shape=[
            jax.ShapeDtypeStruct((_N_EDGES, _D), jnp.float32),
            jax.ShapeDtypeStruct((_N_NODES, _D), jnp.float32),
            jax.ShapeDtypeStruct((_N_NODES, _D), jnp.float32),
        ],
        scratch_shapes=[pltpu.VMEM((_NG, _D), jnp.float32)],
    )(e, W_ee, g, W_eg, b_e2d, edge_idx3d, x, W_es, W_ed)


def _sc_edge_body(psrc, pdst, pe, srcidx, dstidx, gidx,
                  eattr, agg0, agg1, eagg0, eagg1,
                  src_v0, src_v1, src_v2, src_v3,
                  dst_v0, dst_v1, dst_v2, dst_v3,
                  gid_v0, gid_v1, gid_v2, gid_v3,
                  pe_v0, pe_v1, s_v0, s_v1, d_v0, d_v1, o_v0, o_v1,
                  agg_sh, eagg_sh,
                  sem_in0, sem_in1, sem_g0, sem_g1, sem_w0, sem_w1,
                  sem_s0, sem_s1):
    cid = lax.axis_index("c")
    sid = lax.axis_index("s")
    wid = cid * 16 + sid
    src_v = (src_v0, src_v1, src_v2, src_v3)
    dst_v = (dst_v0, dst_v1, dst_v2, dst_v3)
    gid_v = (gid_v0, gid_v1, gid_v2, gid_v3)
    pe_v = (pe_v0, pe_v1)
    s_v = (s_v0, s_v1)
    d_v = (d_v0, d_v1)
    o_v = (o_v0, o_v1)
    sem_in = (sem_in0, sem_in1)
    sem_g = (sem_g0, sem_g1)
    sem_w = (sem_w0, sem_w1)
    sem_s = (sem_s0, sem_s1)

    def _zero_row(i, _):
        for c in range(_D // 16):
            o_v0[i, pl.ds(c * 16, 16)] = jnp.zeros((16,), jnp.float32)
        return 0
    lax.fori_loop(0, _CH, _zero_row, 0)
    nz = _STRIPE // _CH
    rem = _STRIPE - nz * _CH
    for j in range(nz):
        pltpu.sync_copy(o_v0.at[pl.ds(0, _CH)],
                        agg_sh.at[pl.ds(sid * _STRIPE + j * _CH, _CH)])
    if rem:
        pltpu.sync_copy(o_v0.at[pl.ds(0, rem)],
                        agg_sh.at[pl.ds(sid * _STRIPE + nz * _CH, rem)])

    @pl.when(sid == 0)
    def _():
        pltpu.sync_copy(o_v0.at[pl.ds(0, _STRIPE_TAIL)],
                        agg_sh.at[pl.ds(16 * _STRIPE, _STRIPE_TAIL)])
        pltpu.sync_copy(o_v0.at[pl.ds(0, _NG)], eagg_sh)

    plsc.subcore_barrier()

    start = wid * _CH_PER_W
    N = _CH_PER_W

    def _issue_in(t, s2, s4):
        off = (start + t) * _CH
        pltpu.async_copy(srcidx.at[pl.ds(off, _CH)], src_v[s4], sem_in[s2])
        pltpu.async_copy(dstidx.at[pl.ds(off, _CH)], dst_v[s4], sem_in[s2])
        pltpu.async_copy(gidx.at[pl.ds(off, _CH)], gid_v[s4], sem_in[s2])
        pltpu.async_copy(pe.at[pl.ds(off, _CH)], pe_v[s2], sem_in[s2])

    def _wait_in(s2, s4):
        pltpu.make_async_copy(srcidx.at[pl.ds(0, _CH)], src_v[s4], sem_in[s2]).wait()
        pltpu.make_async_copy(dstidx.at[pl.ds(0, _CH)], dst_v[s4], sem_in[s2]).wait()
        pltpu.make_async_copy(gidx.at[pl.ds(0, _CH)], gid_v[s4], sem_in[s2]).wait()
        pltpu.make_async_copy(pe.at[pl.ds(0, _CH)], pe_v[s2], sem_in[s2]).wait()

    def _issue_gather(s2, s4):
        pltpu.async_copy(psrc.at[src_v[s4]], s_v[s2], sem_g[s2])
        pltpu.async_copy(pdst.at[dst_v[s4]], d_v[s2], sem_g[s2])

    def _wait_gather(s2, s4):
        pltpu.make_async_copy(psrc.at[src_v[s4]], s_v[s2], sem_g[s2]).wait()
        pltpu.make_async_copy(pdst.at[dst_v[s4]], d_v[s2], sem_g[s2]).wait()

    def _compute(s2):
        def _row(i, _):
            for c in range(_D // 16):
                sl = pl.ds(c * 16, 16)
                v = pe_v[s2][i, sl] + s_v[s2][i, sl] + d_v[s2][i, sl]
                o_v[s2][i, sl] = jnp.maximum(v, 0.0)
            return 0
        lax.fori_loop(0, _CH, _row, 0)

    def _issue_w(t, s2):
        off = (start + t) * _CH
        pltpu.async_copy(o_v[s2], eattr.at[pl.ds(off, _CH)], sem_w[s2])

    def _wait_w(s2):
        pltpu.make_async_copy(o_v[s2], eattr.at[pl.ds(0, _CH)], sem_w[s2]).wait()

    def _issue_s(s2, s4):
        pltpu.async_copy(o_v[s2], agg_sh.at[dst_v[s4]], sem_s[s2], add=True)
        pltpu.async_copy(o_v[s2], eagg_sh.at[gid_v[s4]], sem_s[s2], add=True)

    def _wait_s(s2, s4):
        pltpu.make_async_copy(o_v[s2], agg_sh.at[dst_v[s4]], sem_s[s2]).wait()
        pltpu.make_async_copy(o_v[s2], eagg_sh.at[gid_v[s4]], sem_s[s2]).wait()

    def _step(t, s2, s4, first2=False, has_next=True, has_next2=True):
        if has_next:
            _wait_in((s2 + 1) % 2, (s4 + 1) % 4)
        _wait_gather(s2, s4)
        if has_next:
            _issue_gather((s2 + 1) % 2, (s4 + 1) % 4)
        if not first2:
            _wait_w(s2)
            _wait_s(s2, (s4 + 2) % 4)
        _compute(s2)
        _issue_w(t, s2)
        _issue_s(s2, s4)
        if has_next2:
            _issue_in(t + 2, s2, (s4 + 2) % 4)

    _issue_in(0, 0, 0)
    _wait_in(0, 0)
    _issue_gather(0, 0)
    _issue_in(1, 1, 1)
    for t in range(4):
        _step(t, t % 2, t % 4, first2=(t < 2))

    def _quad(g, _):
        t0 = 4 * g + 4
        for q in range(4):
            _step(t0 + q, q % 2, q % 4)
        return 0

    lax.fori_loop(0, (N - 6) // 4, _quad, 0)

    _step(N - 2, (N - 2) % 2, (N - 2) % 4, has_next=True, has_next2=False)
    _step(N - 1, (N - 1) % 2, (N - 1) % 4, has_next=False, has_next2=False)
    _wait_w(0)
    _wait_s(0, (N - 2) % 4)
    _wait_w(1)
    _wait_s(1, (N - 1) % 4)
    plsc.subcore_barrier()

    rows = pl.ds(sid * _STRIPE, _STRIPE)
    tail = pl.ds(16 * _STRIPE, _STRIPE_TAIL)

    @pl.when(cid == 0)
    def _():
        pltpu.sync_copy(agg_sh.at[rows], agg0.at[rows])

    @pl.when(cid == 1)
    def _():
        pltpu.sync_copy(agg_sh.at[rows], agg1.at[rows])

    @pl.when((cid == 0) & (sid == 0))
    def _():
        pltpu.sync_copy(agg_sh.at[tail], agg0.at[tail])
        pltpu.sync_copy(eagg_sh, eagg0)

    @pl.when((cid == 1) & (sid == 0))
    def _():
        pltpu.sync_copy(agg_sh.at[tail], agg1.at[tail])
        pltpu.sync_copy(eagg_sh, eagg1)


def _sc_edge(P_src, P_dst, P_e, src, dst, edge_idx):
    mesh = plsc.VectorSubcoreMesh(core_axis_name="c", subcore_axis_name="s")
    f = pl.kernel(
        _sc_edge_body,
        mesh=mesh,
        out_type=[
            jax.ShapeDtypeStruct((_N_EDGES, _D), jnp.float32),
            jax.ShapeDtypeStruct((_N_NODES, _D), jnp.float32),
            jax.ShapeDtypeStruct((_N_NODES, _D), jnp.float32),
            jax.ShapeDtypeStruct((_NG, _D), jnp.float32),
            jax.ShapeDtypeStruct((_NG, _D), jnp.float32),
        ],
        scratch_types=(
            [pltpu.VMEM((_CH,), jnp.int32)] * 12
            + [pltpu.VMEM((_CH, _D), jnp.float32)] * 8
            + [
                pltpu.VMEM_SHARED((_N_NODES, _D), jnp.float32),
                pltpu.VMEM_SHARED((_NG, _D), jnp.float32),
            ]
            + [pltpu.SemaphoreType.DMA] * 8
        ),
    )
    return f(P_src, P_dst, P_e, src, dst, edge_idx)


def _node_body(x_ref, a0_ref, a1_ref, nidx_ref, wn1_ref, wn2_ref,
               g_ref, wn3_ref, bn_ref, e0_ref, e1_ref,
               wg1_ref, wg2_ref, wg3_ref, bg_ref,
               na_ref, ga_ref, gn_scr, gacc_scr):
    i = pl.program_id(0)
    nsteps = pl.num_programs(0)

    @pl.when(i == 0)
    def _():
        gn_scr[...] = (jnp.dot(g_ref[...], wn3_ref[...],
                               preferred_element_type=jnp.float32)
                       + bn_ref[...])
        gacc_scr[...] = jnp.zeros_like(gacc_scr)

    nidx = nidx_ref[0, 0, :]
    nb = nidx.shape[0]
    oh = (nidx[:, None] == lax.broadcasted_iota(jnp.int32, (nb, _NG), 1)
          ).astype(jnp.float32)
    oht = (nidx[None, :] == lax.broadcasted_iota(jnp.int32, (_NG, nb), 0)
           ).astype(jnp.float32)
    agg = a0_ref[...] + a1_ref[...]
    na = (jnp.dot(x_ref[...], wn1_ref[...], preferred_element_type=jnp.float32)
          + jnp.dot(agg, wn2_ref[...], preferred_element_type=jnp.float32)
          + jnp.dot(oh, gn_scr[...], preferred_element_type=jnp.float32))
    na = jnp.maximum(na, 0.0)
    na_ref[...] = na
    gacc_scr[...] += jnp.dot(oht, na, preferred_element_type=jnp.float32)

    @pl.when(i == nsteps - 1)
    def _():
        eagg = e0_ref[...] + e1_ref[...]
        ga_ref[...] = (
            jnp.dot(g_ref[...], wg1_ref[...], preferred_element_type=jnp.float32)
            + jnp.dot(gacc_scr[...], wg2_ref[...], preferred_element_type=jnp.float32)
            + jnp.dot(eagg, wg3_ref[...], preferred_element_type=jnp.float32)
            + bg_ref[...])


def _node_global(x, agg0, agg1, nidx3d, W_n1, W_n2, g, W_n3, b_n2d,
                 eagg0, eagg1, W_g1, W_g2, W_g3, b_g2d):
    blk = 1000
    grid = _N_NODES // blk
    return pl.pallas_call(
        _node_body,
        grid=(grid,),
        in_specs=[
            pl.BlockSpec((blk, _D), lambda i: (i, 0)),
            pl.BlockSpec((blk, _D), lambda i: (i, 0)),
            pl.BlockSpec((blk, _D), lambda i: (i, 0)),
            pl.BlockSpec((1, 1, blk), lambda i: (i, 0, 0)),
            pl.BlockSpec((_D, _D), lambda i: (0, 0)),
            pl.BlockSpec((_D, _D), lambda i: (0, 0)),
            pl.BlockSpec((_NG, 64), lambda i: (0, 0)),
            pl.BlockSpec((64, _D), lambda i: (0, 0)),
            pl.BlockSpec((1, _D), lambda i: (0, 0)),
            pl.BlockSpec((_NG, _D), lambda i: (0, 0)),
            pl.BlockSpec((_NG, _D), lambda i: (0, 0)),
            pl.BlockSpec((64, 64), lambda i: (0, 0)),
            pl.BlockSpec((_D, 64), lambda i: (0, 0)),
            pl.BlockSpec((_D, 64), lambda i: (0, 0)),
            pl.BlockSpec((1, 64), lambda i: (0, 0)),
        ],
        out_specs=[
            pl.BlockSpec((blk, _D), lambda i: (i, 0)),
            pl.BlockSpec((_NG, 64), lambda i: (0, 0)),
        ],
        out_shape=[
            jax.ShapeDtypeStruct((_N_NODES, _D), jnp.float32),
            jax.ShapeDtypeStruct((_NG, 64), jnp.float32),
        ],
        scratch_shapes=[
            pltpu.VMEM((_NG, _D), jnp.float32),
            pltpu.VMEM((_NG, _D), jnp.float32),
        ],
    )(x, agg0, agg1, nidx3d, W_n1, W_n2, g, W_n3, b_n2d,
      eagg0, eagg1, W_g1, W_g2, W_g3, b_g2d)


@jax.jit
def kernel(x, e, g, W_e, b_e, W_n, b_n, W_g, b_g, edges, node_idx, edge_idx):
    src = edges[0].astype(jnp.int32)
    dst = edges[1].astype(jnp.int32)
    edge_idx = edge_idx.astype(jnp.int32)
    node_idx = node_idx.astype(jnp.int32)

    W_ee, W_es, W_ed, W_eg = W_e[:16], W_e[16:144], W_e[144:272], W_e[272:336]
    W_n1, W_n2, W_n3 = W_n[:128], W_n[128:256], W_n[256:320]
    W_g1, W_g2, W_g3 = W_g[:64], W_g[64:192], W_g[192:320]

    P_e, P_src, P_dst = _proj_edges(
        e, W_ee, g, W_eg, b_e.reshape(1, _D),
        edge_idx.reshape(_N_EDGES // 2000, 1, 2000), x, W_es, W_ed)
    edge_attr, agg0, agg1, eagg0, eagg1 = _sc_edge(P_src, P_dst, P_e,
                                                   src, dst, edge_idx)
    node_attr, global_attr = _node_global(
        x, agg0, agg1, node_idx.reshape(_N_NODES // 1000, 1, 1000),
        W_n1, W_n2, g, W_n3, b_n.reshape(1, _D),
        eagg0, eagg1, W_g1, W_g2, W_g3, b_g.reshape(1, 64))
    return (edge_attr, node_attr, global_attr)
